# baseline (speedup 1.0000x reference)
import jax
import jax.numpy as jnp
from jax.experimental import pallas as pl
from functools import partial

_HIDDEN = 4096
_E = 64
_EK = 4
_TOKENS = 8192
_T = 512
_C = 512
_NT = _TOKENS // _T
_NEG = -3.4e38


def _route_chunk(lT, m_ref, r_ref, o):
    C = lT.shape[1]
    riota = jax.lax.broadcasted_iota(jnp.int32, (_E, C), 0)
    v = lT
    sels = []
    vals = []
    for k in range(_EK):
        mv = jnp.max(v, axis=0, keepdims=True)
        idx = jnp.min(jnp.where(v == mv, riota, _E), axis=0, keepdims=True)
        sels.append(idx)
        vals.append(mv)
        if k < _EK - 1:
            v = jnp.where(riota == idx, _NEG, v)
    e1 = jnp.exp(vals[1] - vals[0])
    e2 = jnp.exp(vals[2] - vals[0])
    e3 = jnp.exp(vals[3] - vals[0])
    s = 1.0 + e1 + e2 + e3
    r_ref[:, o:o + C] = jnp.concatenate(
        [1.0 / s, e1 / s, e2 / s, e3 / s], axis=0)
    row = jax.lax.broadcasted_iota(jnp.int32, (4 * _E, C), 0)
    e_row = row >> 2
    b0 = (row & 1) == 1
    b1 = (row & 2) == 2
    s01 = jnp.where(b0, sels[1], sels[0])
    s23 = jnp.where(b0, sels[3], sels[2])
    s_int = jnp.where(b1, s23, s01)
    m_ref[:, o:o + C] = (s_int == e_row).astype(jnp.int32)


def _fused_kernel(x_ref, w_ref, b_ref,
                  m1_ref, m2_ref, m3_ref, r1_ref, r2_ref, r3_ref):
    w = w_ref[...]
    b = b_ref[...]
    for c in range(_T // _C):
        o = c * _C
        lT = jax.lax.dot_general(
            w, x_ref[o:o + _C, :], (((1,), (1,)), ((), ())),
            preferred_element_type=jnp.float32,
        ) + b
        _route_chunk(lT[0:_E, :], m1_ref, r1_ref, o)
        _route_chunk(lT[_E:2 * _E, :], m2_ref, r2_ref, o)
        _route_chunk(lT[2 * _E:3 * _E, :], m3_ref, r3_ref, o)


@jax.jit
def kernel(x, W1, b1, W2, b2, W3, b3):
    x2 = x.reshape(-1, _HIDDEN)
    W = jnp.concatenate([W1, W2, W3], axis=0)
    b = jnp.concatenate([b1, b2, b3], axis=0)[:, None]

    mask_shape = jax.ShapeDtypeStruct((4 * _E, _TOKENS), jnp.int32)
    w_shape = jax.ShapeDtypeStruct((_EK, _TOKENS), jnp.float32)

    outs = pl.pallas_call(
        _fused_kernel,
        grid=(_NT,),
        in_specs=[
            pl.BlockSpec((_T, _HIDDEN), lambda i: (i, 0)),
            pl.BlockSpec((3 * _E, _HIDDEN), lambda i: (0, 0)),
            pl.BlockSpec((3 * _E, 1), lambda i: (0, 0)),
        ],
        out_specs=[
            pl.BlockSpec((4 * _E, _T), lambda i: (0, i)),
            pl.BlockSpec((4 * _E, _T), lambda i: (0, i)),
            pl.BlockSpec((4 * _E, _T), lambda i: (0, i)),
            pl.BlockSpec((_EK, _T), lambda i: (0, i)),
            pl.BlockSpec((_EK, _T), lambda i: (0, i)),
            pl.BlockSpec((_EK, _T), lambda i: (0, i)),
        ],
        out_shape=[mask_shape, mask_shape, mask_shape,
                   w_shape, w_shape, w_shape],
    )(x2, W, b)
    m1, m2, m3, rT1, rT2, rT3 = outs
    shp = (_E, _EK, _TOKENS)
    return (m1.reshape(shp), m2.reshape(shp), m3.reshape(shp),
            rT1.T, rT2.T, rT3.T)


# speedup vs baseline: 1.0254x; 1.0254x over previous
import jax
import jax.numpy as jnp
from jax.experimental import pallas as pl
from functools import partial

_HIDDEN = 4096
_E = 64
_EK = 4
_TOKENS = 8192
_T = 1024
_C = 512
_NT = _TOKENS // _T
_NEG = -3.4e38


def _route_chunk(lT, m_ref, r_ref, o):
    C = lT.shape[1]
    riota = jax.lax.broadcasted_iota(jnp.int32, (_E, C), 0)
    v = lT
    sels = []
    vals = []
    for k in range(_EK):
        mv = jnp.max(v, axis=0, keepdims=True)
        idx = jnp.min(jnp.where(v == mv, riota, _E), axis=0, keepdims=True)
        sels.append(idx)
        vals.append(mv)
        if k < _EK - 1:
            v = jnp.where(riota == idx, _NEG, v)
    e1 = jnp.exp(vals[1] - vals[0])
    e2 = jnp.exp(vals[2] - vals[0])
    e3 = jnp.exp(vals[3] - vals[0])
    s = 1.0 + e1 + e2 + e3
    r_ref[:, o:o + C] = jnp.concatenate(
        [1.0 / s, e1 / s, e2 / s, e3 / s], axis=0)
    row = jax.lax.broadcasted_iota(jnp.int32, (4 * _E, C), 0)
    e_row = row >> 2
    b0 = (row & 1) == 1
    b1 = (row & 2) == 2
    s01 = jnp.where(b0, sels[1], sels[0])
    s23 = jnp.where(b0, sels[3], sels[2])
    s_int = jnp.where(b1, s23, s01)
    m_ref[:, o:o + C] = (s_int == e_row).astype(jnp.int32)


def _fused_kernel(x_ref, w_ref, b_ref,
                  m1_ref, m2_ref, m3_ref, r1_ref, r2_ref, r3_ref):
    w = w_ref[...]
    b = b_ref[...]
    lts = []
    for c in range(_T // _C):
        o = c * _C
        lts.append(jax.lax.dot_general(
            w, x_ref[o:o + _C, :], (((1,), (1,)), ((), ())),
            preferred_element_type=jnp.float32,
        ) + b)
    for c in range(_T // _C):
        o = c * _C
        lT = lts[c]
        _route_chunk(lT[0:_E, :], m1_ref, r1_ref, o)
        _route_chunk(lT[_E:2 * _E, :], m2_ref, r2_ref, o)
        _route_chunk(lT[2 * _E:3 * _E, :], m3_ref, r3_ref, o)


@jax.jit
def kernel(x, W1, b1, W2, b2, W3, b3):
    x2 = x.reshape(-1, _HIDDEN)
    W = jnp.concatenate([W1, W2, W3], axis=0)
    b = jnp.concatenate([b1, b2, b3], axis=0)[:, None]

    mask_shape = jax.ShapeDtypeStruct((4 * _E, _TOKENS), jnp.int32)
    w_shape = jax.ShapeDtypeStruct((_EK, _TOKENS), jnp.float32)

    outs = pl.pallas_call(
        _fused_kernel,
        grid=(_NT,),
        in_specs=[
            pl.BlockSpec((_T, _HIDDEN), lambda i: (i, 0)),
            pl.BlockSpec((3 * _E, _HIDDEN), lambda i: (0, 0)),
            pl.BlockSpec((3 * _E, 1), lambda i: (0, 0)),
        ],
        out_specs=[
            pl.BlockSpec((4 * _E, _T), lambda i: (0, i)),
            pl.BlockSpec((4 * _E, _T), lambda i: (0, i)),
            pl.BlockSpec((4 * _E, _T), lambda i: (0, i)),
            pl.BlockSpec((_EK, _T), lambda i: (0, i)),
            pl.BlockSpec((_EK, _T), lambda i: (0, i)),
            pl.BlockSpec((_EK, _T), lambda i: (0, i)),
        ],
        out_shape=[mask_shape, mask_shape, mask_shape,
                   w_shape, w_shape, w_shape],
    )(x2, W, b)
    m1, m2, m3, rT1, rT2, rT3 = outs
    shp = (_E, _EK, _TOKENS)
    return (m1.reshape(shp), m2.reshape(shp), m3.reshape(shp),
            rT1.T, rT2.T, rT3.T)


# speedup vs baseline: 1.4851x; 1.4483x over previous
import jax
import jax.numpy as jnp
from jax.experimental import pallas as pl
from functools import partial

_HIDDEN = 4096
_E = 64
_EK = 4
_TOKENS = 8192
_T = 1024
_C = 512
_NT = _TOKENS // _T
_NEG = -3.4e38


def _route_chunk(lT, m_ref, r_ref, o):
    C = lT.shape[1]
    riota = jax.lax.broadcasted_iota(jnp.int32, (_E, C), 0)
    v = lT
    sels = []
    vals = []
    for k in range(_EK):
        mv = jnp.max(v, axis=0, keepdims=True)
        idx = jnp.min(jnp.where(v == mv, riota, _E), axis=0, keepdims=True)
        sels.append(idx)
        vals.append(mv)
        if k < _EK - 1:
            v = jnp.where(riota == idx, _NEG, v)
    e1 = jnp.exp(vals[1] - vals[0])
    e2 = jnp.exp(vals[2] - vals[0])
    e3 = jnp.exp(vals[3] - vals[0])
    s = 1.0 + e1 + e2 + e3
    r_ref[:, o:o + C] = jnp.concatenate(
        [1.0 / s, e1 / s, e2 / s, e3 / s], axis=0)
    for k in range(_EK):
        m_ref[:, k, o:o + C] = (sels[k] == riota).astype(jnp.int32)


def _fused_kernel(x_ref, w_ref, b_ref,
                  m1_ref, m2_ref, m3_ref, r1_ref, r2_ref, r3_ref):
    w = w_ref[...]
    b = b_ref[...]
    lts = []
    for c in range(_T // _C):
        o = c * _C
        lts.append(jax.lax.dot_general(
            w, x_ref[o:o + _C, :], (((1,), (1,)), ((), ())),
            preferred_element_type=jnp.float32,
        ) + b)
    for c in range(_T // _C):
        o = c * _C
        lT = lts[c]
        _route_chunk(lT[0:_E, :], m1_ref, r1_ref, o)
        _route_chunk(lT[_E:2 * _E, :], m2_ref, r2_ref, o)
        _route_chunk(lT[2 * _E:3 * _E, :], m3_ref, r3_ref, o)


@jax.jit
def kernel(x, W1, b1, W2, b2, W3, b3):
    x2 = x.reshape(-1, _HIDDEN)
    W = jnp.concatenate([W1, W2, W3], axis=0)
    b = jnp.concatenate([b1, b2, b3], axis=0)[:, None]

    mask_shape = jax.ShapeDtypeStruct((_E, _EK, _TOKENS), jnp.int32)
    w_shape = jax.ShapeDtypeStruct((_EK, _TOKENS), jnp.float32)

    outs = pl.pallas_call(
        _fused_kernel,
        grid=(_NT,),
        in_specs=[
            pl.BlockSpec((_T, _HIDDEN), lambda i: (i, 0)),
            pl.BlockSpec((3 * _E, _HIDDEN), lambda i: (0, 0)),
            pl.BlockSpec((3 * _E, 1), lambda i: (0, 0)),
        ],
        out_specs=[
            pl.BlockSpec((_E, _EK, _T), lambda i: (0, 0, i)),
            pl.BlockSpec((_E, _EK, _T), lambda i: (0, 0, i)),
            pl.BlockSpec((_E, _EK, _T), lambda i: (0, 0, i)),
            pl.BlockSpec((_EK, _T), lambda i: (0, i)),
            pl.BlockSpec((_EK, _T), lambda i: (0, i)),
            pl.BlockSpec((_EK, _T), lambda i: (0, i)),
        ],
        out_shape=[mask_shape, mask_shape, mask_shape,
                   w_shape, w_shape, w_shape],
    )(x2, W, b)
    m1, m2, m3, rT1, rT2, rT3 = outs
    return (m1, m2, m3, rT1.T, rT2.T, rT3.T)


# speedup vs baseline: 1.5180x; 1.0221x over previous
import jax
import jax.numpy as jnp
from jax.experimental import pallas as pl
from functools import partial

_HIDDEN = 4096
_E = 64
_EK = 4
_TOKENS = 8192
_T = 1024
_C = 1024
_NT = _TOKENS // _T
_NEG = -3.4e38


def _route_chunk(lT, m_ref, r_ref, o):
    C = lT.shape[1]
    riota = jax.lax.broadcasted_iota(jnp.int32, (_E, C), 0)
    v = lT
    sels = []
    vals = []
    for k in range(_EK):
        mv = jnp.max(v, axis=0, keepdims=True)
        idx = jnp.min(jnp.where(v == mv, riota, _E), axis=0, keepdims=True)
        sels.append(idx)
        vals.append(mv)
        if k < _EK - 1:
            v = jnp.where(riota == idx, _NEG, v)
    e1 = jnp.exp(vals[1] - vals[0])
    e2 = jnp.exp(vals[2] - vals[0])
    e3 = jnp.exp(vals[3] - vals[0])
    s = 1.0 + e1 + e2 + e3
    r_ref[:, o:o + C] = jnp.concatenate(
        [1.0 / s, e1 / s, e2 / s, e3 / s], axis=0)
    for k in range(_EK):
        m_ref[:, k, o:o + C] = (sels[k] == riota).astype(jnp.int32)


def _fused_kernel(x_ref, w_ref, b_ref,
                  m1_ref, m2_ref, m3_ref, r1_ref, r2_ref, r3_ref):
    w = w_ref[...]
    b = b_ref[...]
    lts = []
    for c in range(_T // _C):
        o = c * _C
        lts.append(jax.lax.dot_general(
            w, x_ref[o:o + _C, :], (((1,), (1,)), ((), ())),
            preferred_element_type=jnp.float32,
        ) + b)
    for c in range(_T // _C):
        o = c * _C
        lT = lts[c]
        _route_chunk(lT[0:_E, :], m1_ref, r1_ref, o)
        _route_chunk(lT[_E:2 * _E, :], m2_ref, r2_ref, o)
        _route_chunk(lT[2 * _E:3 * _E, :], m3_ref, r3_ref, o)


@jax.jit
def kernel(x, W1, b1, W2, b2, W3, b3):
    x2 = x.reshape(-1, _HIDDEN)
    W = jnp.concatenate([W1, W2, W3], axis=0)
    b = jnp.concatenate([b1, b2, b3], axis=0)[:, None]

    mask_shape = jax.ShapeDtypeStruct((_E, _EK, _TOKENS), jnp.int32)
    w_shape = jax.ShapeDtypeStruct((_EK, _TOKENS), jnp.float32)

    outs = pl.pallas_call(
        _fused_kernel,
        grid=(_NT,),
        in_specs=[
            pl.BlockSpec((_T, _HIDDEN), lambda i: (i, 0)),
            pl.BlockSpec((3 * _E, _HIDDEN), lambda i: (0, 0)),
            pl.BlockSpec((3 * _E, 1), lambda i: (0, 0)),
        ],
        out_specs=[
            pl.BlockSpec((_E, _EK, _T), lambda i: (0, 0, i)),
            pl.BlockSpec((_E, _EK, _T), lambda i: (0, 0, i)),
            pl.BlockSpec((_E, _EK, _T), lambda i: (0, 0, i)),
            pl.BlockSpec((_EK, _T), lambda i: (0, i)),
            pl.BlockSpec((_EK, _T), lambda i: (0, i)),
            pl.BlockSpec((_EK, _T), lambda i: (0, i)),
        ],
        out_shape=[mask_shape, mask_shape, mask_shape,
                   w_shape, w_shape, w_shape],
    )(x2, W, b)
    m1, m2, m3, rT1, rT2, rT3 = outs
    return (m1, m2, m3, rT1.T, rT2.T, rT3.T)
